# baseline (speedup 1.0000x reference)
import functools

import jax
import jax.numpy as jnp
from jax.experimental import pallas as pl
from jax.experimental.pallas import tpu as pltpu

_EPS = 1e-5


def _bdot(a, b):
    return jnp.dot(a.astype(jnp.bfloat16), b.astype(jnp.bfloat16),
                   preferred_element_type=jnp.float32)


def _proj_kernel(x_ref, w_ref, out_ref):
    out_ref[...] = _bdot(x_ref[...], w_ref[...])


def _main1_kernel(sup_ref, a_ref, b_ref, out_ref, stats_ref, supout_ref):
    sup = sup_ref[...]
    r = jnp.maximum(_bdot(sup, a_ref[...]) + b_ref[...], 0.0)
    out_ref[...] = r
    stats_ref[0, 0, :] = jnp.sum(r, axis=0)
    stats_ref[0, 1, :] = jnp.sum(r * r, axis=0)
    supout_ref[...] = sup


def _main2_kernel(sup_ref, a_ref, b_ref, out_ref, stats_ref):
    r = jnp.maximum(_bdot(sup_ref[...], a_ref[...]) + b_ref[...], 0.0)
    out_ref[...] = r
    stats_ref[0, 0, :] = jnp.sum(r, axis=0)
    stats_ref[0, 1, :] = jnp.sum(r * r, axis=0)


def _norm_kernel(r_ref, stats_ref, gamma_ref, beta_ref, w_ref, out_ref, *, n):
    s = jnp.sum(stats_ref[:, 0, :], axis=0)
    s2 = jnp.sum(stats_ref[:, 1, :], axis=0)
    mu = s / n
    var = s2 / n - mu * mu
    scale = gamma_ref[0, :] / jnp.sqrt(var + _EPS)
    shift = beta_ref[0, :] - mu * scale
    h = r_ref[...] * scale[None, :] + shift[None, :]
    if w_ref is not None:
        h = _bdot(h, w_ref[...])
    out_ref[...] = h


def _norm_kernel_now(r_ref, stats_ref, gamma_ref, beta_ref, out_ref, *, n):
    _norm_kernel(r_ref, stats_ref, gamma_ref, beta_ref, None, out_ref, n=n)


def _proj(x, w):
    n, _ = x.shape
    d = w.shape[1]
    return pl.pallas_call(
        _proj_kernel,
        out_shape=jax.ShapeDtypeStruct((n, d), jnp.float32),
    )(x, w)


def _main_pass(sup, a, b, bm, copy_sup):
    n = sup.shape[0]
    d = a.shape[1]
    g = n // bm
    out_specs = [
        pl.BlockSpec((bm, d), lambda i: (i, 0)),
        pl.BlockSpec((1, 2, d), lambda i: (i, 0, 0)),
    ]
    out_shape = [
        jax.ShapeDtypeStruct((n, d), jnp.float32),
        jax.ShapeDtypeStruct((g, 2, d), jnp.float32),
    ]
    if copy_sup:
        out_specs.append(pl.BlockSpec((bm, n), lambda i: (i, 0)))
        out_shape.append(jax.ShapeDtypeStruct((n, n), jnp.float32))
    return pl.pallas_call(
        _main1_kernel if copy_sup else _main2_kernel,
        grid=(g,),
        in_specs=[
            pl.BlockSpec((bm, n), lambda i: (i, 0)),
            pl.BlockSpec((n, d), lambda i: (0, 0)),
            pl.BlockSpec((1, d), lambda i: (0, 0)),
        ],
        out_specs=out_specs,
        out_shape=out_shape,
        compiler_params=pltpu.CompilerParams(
            dimension_semantics=("parallel",)),
    )(sup, a, b.reshape(1, d))


def _norm_pass(r, stats, gamma, beta, w, bm):
    n, d = r.shape
    d_out = w.shape[1] if w is not None else d
    g = n // bm
    in_specs = [
        pl.BlockSpec((bm, d), lambda i: (i, 0)),
        pl.BlockSpec(stats.shape, lambda i: (0, 0, 0)),
        pl.BlockSpec((1, d), lambda i: (0, 0)),
        pl.BlockSpec((1, d), lambda i: (0, 0)),
    ]
    args = [r, stats, gamma.reshape(1, d), beta.reshape(1, d)]
    if w is not None:
        in_specs.append(pl.BlockSpec(w.shape, lambda i: (0, 0)))
        args.append(w)
        body = functools.partial(_norm_kernel, n=float(n))
    else:
        body = functools.partial(_norm_kernel_now, n=float(n))
    return pl.pallas_call(
        body,
        grid=(g,),
        in_specs=in_specs,
        out_specs=pl.BlockSpec((bm, d_out), lambda i: (i, 0)),
        out_shape=jax.ShapeDtypeStruct((n, d_out), jnp.float32),
        compiler_params=pltpu.CompilerParams(
            dimension_semantics=("parallel",)),
    )(*args)


def _pick_block(n, target):
    best = 8
    for cand in range(8, min(n, target) + 1, 8):
        if n % cand == 0:
            best = cand
    return best


def kernel(x, support, W1, b1, gamma1, beta1, W2, b2, gamma2, beta2):
    n = support.shape[0]
    bm1 = _pick_block(n, 200)
    bm2 = _pick_block(n, 400)
    bm_norm = _pick_block(n, 2000)
    a = _proj(x, W1)
    r1, stats1, sup_out = _main_pass(support, a, b1, bm1, True)
    bmat = _norm_pass(r1, stats1, gamma1, beta1, W2, bm_norm)
    r2, stats2 = _main_pass(support, bmat, b2, bm2, False)
    out = _norm_pass(r2, stats2, gamma2, beta2, None, bm_norm)
    return (out, sup_out)


# speedup vs baseline: 1.0131x; 1.0131x over previous
import functools

import jax
import jax.numpy as jnp
from jax.experimental import pallas as pl
from jax.experimental.pallas import tpu as pltpu

_EPS = 1e-5


def _bdot(a, b):
    return jnp.dot(a.astype(jnp.bfloat16), b.astype(jnp.bfloat16),
                   preferred_element_type=jnp.float32)


def _proj_kernel(x_ref, w_ref, out_ref):
    out_ref[...] = _bdot(x_ref[...], w_ref[...]).astype(jnp.bfloat16)


def _main1_kernel(sup_ref, a_ref, b_ref, out_ref, stats_ref, supout_ref):
    sup = sup_ref[...]
    r = jnp.maximum(_bdot(sup, a_ref[...]) + b_ref[...], 0.0)
    out_ref[...] = r
    stats_ref[0, 0, :] = jnp.sum(r, axis=0)
    stats_ref[0, 1, :] = jnp.sum(r * r, axis=0)
    supout_ref[...] = sup


def _main2_kernel(sup_ref, a_ref, b_ref, out_ref, stats_ref):
    r = jnp.maximum(_bdot(sup_ref[...], a_ref[...]) + b_ref[...], 0.0)
    out_ref[...] = r
    stats_ref[0, 0, :] = jnp.sum(r, axis=0)
    stats_ref[0, 1, :] = jnp.sum(r * r, axis=0)


def _norm_kernel(r_ref, stats_ref, gamma_ref, beta_ref, w_ref, out_ref, *, n):
    s = jnp.sum(stats_ref[:, 0, :], axis=0)
    s2 = jnp.sum(stats_ref[:, 1, :], axis=0)
    mu = s / n
    var = s2 / n - mu * mu
    scale = gamma_ref[0, :] / jnp.sqrt(var + _EPS)
    shift = beta_ref[0, :] - mu * scale
    h = r_ref[...] * scale[None, :] + shift[None, :]
    if w_ref is not None:
        h = _bdot(h, w_ref[...]).astype(jnp.bfloat16)
    out_ref[...] = h


def _norm_kernel_now(r_ref, stats_ref, gamma_ref, beta_ref, out_ref, *, n):
    _norm_kernel(r_ref, stats_ref, gamma_ref, beta_ref, None, out_ref, n=n)


def _proj(x, w):
    n, _ = x.shape
    d = w.shape[1]
    return pl.pallas_call(
        _proj_kernel,
        out_shape=jax.ShapeDtypeStruct((n, d), jnp.bfloat16),
    )(x, w)


def _main_pass(sup, a, b, bm, copy_sup):
    n = sup.shape[0]
    d = a.shape[1]
    g = n // bm
    out_specs = [
        pl.BlockSpec((bm, d), lambda i: (i, 0)),
        pl.BlockSpec((1, 2, d), lambda i: (i, 0, 0)),
    ]
    out_shape = [
        jax.ShapeDtypeStruct((n, d), jnp.float32),
        jax.ShapeDtypeStruct((g, 2, d), jnp.float32),
    ]
    if copy_sup:
        out_specs.append(pl.BlockSpec((bm, n), lambda i: (i, 0)))
        out_shape.append(jax.ShapeDtypeStruct((n, n), jnp.float32))
    return pl.pallas_call(
        _main1_kernel if copy_sup else _main2_kernel,
        grid=(g,),
        in_specs=[
            pl.BlockSpec((bm, n), lambda i: (i, 0)),
            pl.BlockSpec((n, d), lambda i: (0, 0)),
            pl.BlockSpec((1, d), lambda i: (0, 0)),
        ],
        out_specs=out_specs,
        out_shape=out_shape,
        compiler_params=pltpu.CompilerParams(
            dimension_semantics=("parallel",)),
    )(sup, a, b.reshape(1, d))


def _norm_pass(r, stats, gamma, beta, w, bm):
    n, d = r.shape
    d_out = w.shape[1] if w is not None else d
    g = n // bm
    in_specs = [
        pl.BlockSpec((bm, d), lambda i: (i, 0)),
        pl.BlockSpec(stats.shape, lambda i: (0, 0, 0)),
        pl.BlockSpec((1, d), lambda i: (0, 0)),
        pl.BlockSpec((1, d), lambda i: (0, 0)),
    ]
    args = [r, stats, gamma.reshape(1, d), beta.reshape(1, d)]
    if w is not None:
        in_specs.append(pl.BlockSpec(w.shape, lambda i: (0, 0)))
        args.append(w)
        body = functools.partial(_norm_kernel, n=float(n))
    else:
        body = functools.partial(_norm_kernel_now, n=float(n))
    out_dtype = jnp.bfloat16 if w is not None else jnp.float32
    return pl.pallas_call(
        body,
        grid=(g,),
        in_specs=in_specs,
        out_specs=pl.BlockSpec((bm, d_out), lambda i: (i, 0)),
        out_shape=jax.ShapeDtypeStruct((n, d_out), out_dtype),
        compiler_params=pltpu.CompilerParams(
            dimension_semantics=("parallel",)),
    )(*args)


def _pick_block(n, target):
    best = 8
    for cand in range(8, min(n, target) + 1, 8):
        if n % cand == 0:
            best = cand
    return best


def kernel(x, support, W1, b1, gamma1, beta1, W2, b2, gamma2, beta2):
    n = support.shape[0]
    bm1 = _pick_block(n, 200)
    bm2 = _pick_block(n, 400)
    bm_norm = _pick_block(n, 2000)
    a = _proj(x, W1)
    r1, stats1, sup_out = _main_pass(support, a, b1, bm1, True)
    bmat = _norm_pass(r1, stats1, gamma1, beta1, W2, bm_norm)
    r2, stats2 = _main_pass(support, bmat, b2, bm2, False)
    out = _norm_pass(r2, stats2, gamma2, beta2, None, bm_norm)
    return (out, sup_out)
